# baseline (speedup 1.0000x reference)
import jax
import jax.numpy as jnp
from jax import lax
from jax.experimental import pallas as pl
from jax.experimental.pallas import tpu as pltpu
from jax.experimental.pallas import tpu_sc as plsc

B = 16384
F = 64
NC = 2
NS = 16
NW = NC * NS
BPW = B // NW
STRIDE = 384
W = 512
QMAX = 2603
DUMP = B
GROUPS = BPW // 16

_CPARAMS = pltpu.CompilerParams(
    needs_layout_passes=False, disable_bounds_checks=True)


def _qchunk(v):
    return jnp.minimum(
        lax.shift_right_logical(
            lax.shift_right_logical(v, 7) * 43691, 17).astype(jnp.int32),
        QMAX)


def _sweep(sv_hbm, pv_hbm, tabT_hbm, rows_hbm, sv, pv, slabs, stage,
           sems, semws, base, lane):
    pltpu.sync_copy(sv_hbm.at[pl.ds(base, BPW)], sv.at[pl.ds(0, BPW)])
    pltpu.sync_copy(pv_hbm.at[pl.ds(base, BPW)], pv.at[pl.ds(0, BPW)])
    sv[pl.ds(BPW, 16)] = jnp.full((16,), 999999, jnp.int32)
    pv[pl.ds(BPW, 16)] = jnp.full((16,), DUMP, jnp.int32)

    def issue(m, side):
        v = sv[pl.ds(jnp.minimum(m, BPW), 16)]
        c = _qchunk(v)[0]
        col0 = pl.multiple_of(c * STRIDE, 128)
        pltpu.async_copy(tabT_hbm.at[:, pl.ds(col0, W)], slabs[side],
                         sems[side])

    def adv(m):
        v = sv[pl.ds(jnp.minimum(m, BPW), 16)]
        qv = _qchunk(v)
        cnt = plsc.all_reduce_population_count(qv == qv[0])[0]
        return jnp.minimum(m + jnp.maximum(cnt, 1), BPW)

    def prime_writes(side):
        for j in range(16):
            pltpu.async_copy(
                stage.at[pl.ds(side * 1024 + j * F, F)],
                rows_hbm.at[pl.ds(DUMP * F, F)], semws[side])

    def drain_writes(side):
        for j in range(16):
            pltpu.make_async_copy(
                stage.at[pl.ds(side * 1024 + j * F, F)],
                rows_hbm.at[pl.ds(DUMP * F, F)], semws[side]).wait()

    def extract(m, side):
        pltpu.make_async_copy(
            tabT_hbm.at[:, pl.ds(0, W)], slabs[side], sems[side]).wait()
        drain_writes(side)
        mm = jnp.minimum(m, BPW)
        v = sv[pl.ds(mm, 16)]
        bvec = pv[pl.ds(mm, 16)]
        qv = _qchunk(v)
        c = qv[0]
        col_base = c * STRIDE
        for j in range(16):
            active = qv[j] == c
            b = jnp.where(active, bvec[j], DUMP)
            col = jnp.where(active, v[j] - col_base, 0)
            cvec = jnp.full((16,), 0, jnp.int32) + col
            for fg in range(F // 16):
                f_idx = fg * 16 + lane
                vals = plsc.load_gather(slabs[side], [f_idx, cvec])
                stage[pl.ds(side * 1024 + j * F + fg * 16, 16)] = vals
            pltpu.async_copy(
                stage.at[pl.ds(side * 1024 + j * F, F)],
                rows_hbm.at[pl.ds(b * F, F)], semws[side])

    prime_writes(0)
    prime_writes(1)
    m0 = jnp.int32(0)
    issue(m0, 0)
    m1 = adv(m0)
    issue(m1, 1)

    def w_body(carry):
        m0, m1 = carry
        m2 = adv(m1)
        extract(m0, 0)
        issue(m2, 0)
        m3 = adv(m2)
        extract(m1, 1)
        issue(m3, 1)
        return (m2, m3)

    lax.while_loop(lambda cr: cr[0] < BPW, w_body, (m0, m1))
    for side in range(2):
        pltpu.make_async_copy(
            tabT_hbm.at[:, pl.ds(0, W)], slabs[side], sems[side]).wait()
        drain_writes(side)


def _sweep_body(su_hbm, pu_hbm, si_hbm, pi_hbm, uft_hbm, ift_hbm,
                rows_u_hbm, rows_i_hbm,
                sv, pv, slab0, slab1, stage, sem0, sem1, semw0, semw1):
    c = lax.axis_index("c")
    s = lax.axis_index("s")
    base = (s * NC + c) * BPW
    lane = lax.iota(jnp.int32, 16)
    slabs = (slab0, slab1)
    sems = (sem0, sem1)
    semws = (semw0, semw1)
    _sweep(su_hbm, pu_hbm, uft_hbm, rows_u_hbm, sv, pv, slabs, stage,
           sems, semws, base, lane)
    _sweep(si_hbm, pi_hbm, ift_hbm, rows_i_hbm, sv, pv, slabs, stage,
           sems, semws, base, lane)


def _dot_body(rows_u_hbm, rows_i_hbm, out_hbm, ru, ri, out_v, sem):
    c = lax.axis_index("c")
    s = lax.axis_index("s")
    base = (s * NC + c) * BPW
    lane = lax.iota(jnp.int32, 16)
    cp_u = pltpu.async_copy(rows_u_hbm.at[pl.ds(base * F, BPW * F)], ru, sem)
    cp_i = pltpu.async_copy(rows_i_hbm.at[pl.ds(base * F, BPW * F)], ri, sem)
    cp_u.wait()
    cp_i.wait()

    def g_body(g, carry):
        rbase = (g * 16 + lane) * F
        acc = jnp.zeros((16,), jnp.float32)
        for f in range(F):
            u = plsc.load_gather(ru, [rbase + f])
            v = plsc.load_gather(ri, [rbase + f])
            acc = acc + u * v
        out_v[pl.ds(g * 16, 16)] = acc
        return carry

    lax.fori_loop(0, GROUPS, g_body, 0)
    pltpu.sync_copy(out_v, out_hbm.at[pl.ds(base, BPW)])


@jax.jit
def kernel(user, item, user_factors, item_factors):
    mesh = plsc.VectorSubcoreMesh(core_axis_name="c", subcore_axis_name="s")
    sweep = pl.kernel(
        _sweep_body,
        mesh=mesh,
        compiler_params=_CPARAMS,
        out_type=(jax.ShapeDtypeStruct(((B + 1) * F,), jnp.float32),
                  jax.ShapeDtypeStruct(((B + 1) * F,), jnp.float32)),
        scratch_types=[
            pltpu.VMEM((BPW + 16,), jnp.int32),
            pltpu.VMEM((BPW + 16,), jnp.int32),
            pltpu.VMEM((F, W), jnp.float32),
            pltpu.VMEM((F, W), jnp.float32),
            pltpu.VMEM((2048,), jnp.float32),
            pltpu.SemaphoreType.DMA,
            pltpu.SemaphoreType.DMA,
            pltpu.SemaphoreType.DMA,
            pltpu.SemaphoreType.DMA,
        ],
    )
    dot = pl.kernel(
        _dot_body,
        mesh=mesh,
        compiler_params=_CPARAMS,
        out_type=jax.ShapeDtypeStruct((B,), jnp.float32),
        scratch_types=[
            pltpu.VMEM((BPW * F,), jnp.float32),
            pltpu.VMEM((BPW * F,), jnp.float32),
            pltpu.VMEM((BPW,), jnp.float32),
            pltpu.SemaphoreType.DMA,
        ],
    )
    user = user.astype(jnp.int32)
    item = item.astype(jnp.int32)
    pu = jnp.argsort(user).astype(jnp.int32)
    su = user[pu]
    pi = jnp.argsort(item).astype(jnp.int32)
    si = item[pi]
    rows_u, rows_i = sweep(su, pu, si, pi, user_factors.T, item_factors.T)
    return dot(rows_u, rows_i)


# speedup vs baseline: 5.2358x; 5.2358x over previous
import jax
import jax.numpy as jnp
from jax import lax
from jax.experimental import pallas as pl
from jax.experimental.pallas import tpu as pltpu
from jax.experimental.pallas import tpu_sc as plsc

B = 16384
F = 64
RW = 128
NC = 2
NS = 16
NW = NC * NS
BPW = B // NW
W = 512
COL0MAX = 999552
BPWS = BPW + 16
NROWS = NW * BPWS + 32
DUMPROW = NW * BPWS
CHUNK = 128
GROUPS = BPW // 16

_CPARAMS = pltpu.CompilerParams(
    needs_layout_passes=False, disable_bounds_checks=True)


def _sweep(sv_hbm, tabT_hbm, rows_hbm, sv, slabs, stage,
           sems, sem_w, base, rbase, lane):
    pltpu.sync_copy(sv_hbm.at[pl.ds(base, BPW)], sv.at[pl.ds(0, BPW)])
    sv[pl.ds(BPW, 16)] = jnp.full((16,), 999999, jnp.int32)

    def issue(m, side):
        v = sv[pl.ds(jnp.minimum(m, BPW), 16)]
        col0 = pl.multiple_of(
            jnp.minimum(lax.shift_right_logical(v[0], 9) * W, COL0MAX), 128)
        pltpu.async_copy(tabT_hbm.at[:, pl.ds(col0, W)], slabs[side],
                         sems[side])

    def adv(m):
        v = sv[pl.ds(jnp.minimum(m, BPW), 16)]
        qv = lax.shift_right_logical(v, 9)
        cnt = plsc.all_reduce_population_count(qv == qv[0])[0]
        return jnp.minimum(m + jnp.maximum(cnt, 1), BPW)

    def extract(m, side):
        pltpu.make_async_copy(
            tabT_hbm.at[:, pl.ds(0, W)], slabs[side], sems[side]).wait()
        mm = jnp.minimum(m, BPW)
        v = sv[pl.ds(mm, 16)]
        qv = lax.shift_right_logical(v, 9)
        c = qv[0]
        col0 = jnp.minimum(c * W, COL0MAX)
        for j in range(16):
            col = jnp.where(qv[j] == c, v[j] - col0, 0)
            cvec = jnp.full((16,), 0, jnp.int32) + col
            for fg in range(F // 16):
                f_idx = fg * 16 + lane
                vals = plsc.load_gather(slabs[side], [f_idx, cvec])
                stage[pl.ds(j * RW + fg * 16, 16)] = vals
        pltpu.make_async_copy(
            stage, rows_hbm.at[pl.ds(DUMPROW * RW, 16 * RW)], sem_w).wait()
        pltpu.async_copy(
            stage, rows_hbm.at[pl.ds((rbase + mm) * RW, 16 * RW)], sem_w)

    pltpu.async_copy(stage, rows_hbm.at[pl.ds(DUMPROW * RW, 16 * RW)], sem_w)

    m0 = jnp.int32(0)
    issue(m0, 0)
    m1 = adv(m0)
    issue(m1, 1)

    def w_body(carry):
        m0, m1 = carry
        m2 = adv(m1)
        extract(m0, 0)
        issue(m2, 0)
        m3 = adv(m2)
        extract(m1, 1)
        issue(m3, 1)
        return (m2, m3)

    lax.while_loop(lambda cr: cr[0] < BPW, w_body, (m0, m1))
    for side in range(2):
        pltpu.make_async_copy(
            tabT_hbm.at[:, pl.ds(0, W)], slabs[side], sems[side]).wait()
    pltpu.make_async_copy(
        stage, rows_hbm.at[pl.ds(DUMPROW * RW, 16 * RW)], sem_w).wait()


def _sweep_body(su_hbm, si_hbm, uft_hbm, ift_hbm,
                rows_u_hbm, rows_i_hbm,
                sv, slab0, slab1, stage, sem0, sem1, sem_w):
    c = lax.axis_index("c")
    s = lax.axis_index("s")
    wid = s * NC + c
    base = wid * BPW
    rbase = wid * BPWS
    lane = lax.iota(jnp.int32, 16)
    slabs = (slab0, slab1)
    sems = (sem0, sem1)
    _sweep(su_hbm, uft_hbm, rows_u_hbm, sv, slabs, stage,
           sems, sem_w, base, rbase, lane)
    _sweep(si_hbm, ift_hbm, rows_i_hbm, sv, slabs, stage,
           sems, sem_w, base, rbase, lane)


def _dot_body(invu_hbm, invi_hbm, rowsu2_hbm, rowsi2_hbm, out_hbm,
              invu_v, invi_v, blk_u, blk_i, out_v, sem_u, sem_i):
    c = lax.axis_index("c")
    s = lax.axis_index("s")
    base = (s * NC + c) * BPW
    lane = lax.iota(jnp.int32, 16)
    pltpu.sync_copy(invu_hbm.at[pl.ds(base, BPW)], invu_v)
    pltpu.sync_copy(invi_hbm.at[pl.ds(base, BPW)], invi_v)

    def chunk_body(k, carry):
        cp_u = pltpu.async_copy(
            rowsu2_hbm.at[invu_v.at[pl.ds(k * CHUNK, CHUNK)]], blk_u, sem_u)
        cp_i = pltpu.async_copy(
            rowsi2_hbm.at[invi_v.at[pl.ds(k * CHUNK, CHUNK)]], blk_i, sem_i)
        cp_u.wait()
        cp_i.wait()

        def g_body(g, carry2):
            row_b = g * 16 + lane
            acc = jnp.zeros((16,), jnp.float32)
            for f in range(F):
                fv = jnp.full((16,), f, jnp.int32)
                u = plsc.load_gather(blk_u, [row_b, fv])
                v = plsc.load_gather(blk_i, [row_b, fv])
                acc = acc + u * v
            out_v[pl.ds(k * CHUNK + g * 16, 16)] = acc
            return carry2

        lax.fori_loop(0, CHUNK // 16, g_body, 0)
        return carry

    lax.fori_loop(0, BPW // CHUNK, chunk_body, 0)
    pltpu.sync_copy(out_v, out_hbm.at[pl.ds(base, BPW)])


@jax.jit
def kernel(user, item, user_factors, item_factors):
    mesh = plsc.VectorSubcoreMesh(core_axis_name="c", subcore_axis_name="s")
    sweep = pl.kernel(
        _sweep_body,
        mesh=mesh,
        compiler_params=_CPARAMS,
        out_type=(jax.ShapeDtypeStruct((NROWS * RW,), jnp.float32),
                  jax.ShapeDtypeStruct((NROWS * RW,), jnp.float32)),
        scratch_types=[
            pltpu.VMEM((BPW + 16,), jnp.int32),
            pltpu.VMEM((F, W), jnp.float32),
            pltpu.VMEM((F, W), jnp.float32),
            pltpu.VMEM((16 * RW,), jnp.float32),
            pltpu.SemaphoreType.DMA,
            pltpu.SemaphoreType.DMA,
            pltpu.SemaphoreType.DMA,
        ],
    )
    dot = pl.kernel(
        _dot_body,
        mesh=mesh,
        compiler_params=_CPARAMS,
        out_type=jax.ShapeDtypeStruct((B,), jnp.float32),
        scratch_types=[
            pltpu.VMEM((BPW,), jnp.int32),
            pltpu.VMEM((BPW,), jnp.int32),
            pltpu.VMEM((CHUNK, RW), jnp.float32),
            pltpu.VMEM((CHUNK, RW), jnp.float32),
            pltpu.VMEM((BPW,), jnp.float32),
            pltpu.SemaphoreType.DMA,
            pltpu.SemaphoreType.DMA,
        ],
    )
    user = user.astype(jnp.int32)
    item = item.astype(jnp.int32)
    pu = jnp.argsort(user).astype(jnp.int32)
    su = user[pu]
    inv_u = jnp.argsort(pu).astype(jnp.int32)
    inv_u = inv_u + (inv_u // BPW) * 16
    pi = jnp.argsort(item).astype(jnp.int32)
    si = item[pi]
    inv_i = jnp.argsort(pi).astype(jnp.int32)
    inv_i = inv_i + (inv_i // BPW) * 16
    rows_u, rows_i = sweep(su, si, user_factors.T, item_factors.T)
    ru2 = rows_u.reshape(NROWS, RW)
    ri2 = rows_i.reshape(NROWS, RW)
    return dot(inv_u, inv_i, ru2, ri2)
